# baseline (speedup 1.0000x reference)
import functools

import jax
import jax.numpy as jnp
from jax import lax
from jax.experimental import pallas as pl
from jax.experimental.pallas import tpu as pltpu
from jax.experimental.pallas import tpu_sc as plsc

NUM_CORES = 2
NUM_SUBCORES = 16
NW = NUM_CORES * NUM_SUBCORES


def _make_lookup(R, C, D):
    rows_w = R // NW
    G = 400
    n = rows_w * C // G
    mesh = plsc.VectorSubcoreMesh(core_axis_name="c", subcore_axis_name="s")

    @functools.partial(
        pl.kernel,
        out_type=jax.ShapeDtypeStruct((R, C, D), jnp.float32),
        mesh=mesh,
        scratch_types=[
            pltpu.VMEM((rows_w, C), jnp.int32),
            pltpu.VMEM((rows_w * C,), jnp.int32),
            pltpu.VMEM((G, D), jnp.float32),
            pltpu.SemaphoreType.DMA,
        ],
        compiler_params=pltpu.CompilerParams(
            use_tc_tiling_on_sc=False, needs_layout_passes=False),
    )
    def lookup(ids_hbm, w_hbm, out_hbm, idx2d, idx_flat, rows_v, sem):
        wid = lax.axis_index("s") * NUM_CORES + lax.axis_index("c")
        base = wid * rows_w
        pltpu.sync_copy(ids_hbm.at[pl.ds(base, rows_w)], idx2d)

        @pl.loop(0, rows_w * C // 16)
        def _repack(k):
            o = k * 16 + lax.iota(jnp.int32, 16)
            r = (o * 41944) >> 21
            c = o - r * C
            v = plsc.load_gather(idx2d, [r, c])
            idx_flat[pl.ds(k * 16, 16)] = v

        @pl.loop(0, n)
        def _chunk(j):
            idx = idx_flat.at[pl.ds(j * G, G)]
            pltpu.async_copy(w_hbm.at[idx], rows_v, sem).wait()
            for r in range(G // C):
                pltpu.async_copy(
                    rows_v.at[pl.ds(r * C, C)],
                    out_hbm.at[base + j * (G // C) + r],
                    sem,
                ).wait()

    return lookup


def kernel(token_ids, W):
    R, C = token_ids.shape
    D = W.shape[1]
    return _make_lookup(R, C, D)(token_ids.astype(jnp.int32), W)


# speedup vs baseline: 1.4246x; 1.4246x over previous
import functools

import jax
import jax.numpy as jnp
from jax import lax
from jax.experimental import pallas as pl
from jax.experimental.pallas import tpu as pltpu
from jax.experimental.pallas import tpu_sc as plsc

NUM_CORES = 2
NUM_SUBCORES = 16
NW = NUM_CORES * NUM_SUBCORES

TOK_R = 8
K = 2
S = 1
NBUF = K + S

PAD_C = 56
PAD_D = 128


def _make_lookup(R, C, D):
    assert R % (NW * TOK_R) == 0
    rows_w = R // NW
    n = rows_w // TOK_R
    G = TOK_R * C
    mesh = plsc.VectorSubcoreMesh(core_axis_name="c", subcore_axis_name="s")

    @functools.partial(
        pl.kernel,
        out_type=jax.ShapeDtypeStruct((R, PAD_C, PAD_D), jnp.float32),
        mesh=mesh,
        scratch_types=(
            [pltpu.VMEM((rows_w * C,), jnp.int32),
             pltpu.VMEM((NBUF, G, D), jnp.float32)]
            + [pltpu.SemaphoreType.DMA] * (2 * NBUF)
        ),
        compiler_params=pltpu.CompilerParams(
            use_tc_tiling_on_sc=False, needs_layout_passes=False),
    )
    def lookup(ids_hbm, w_hbm, out_hbm, idx_v, rows_v, *sems):
        sem_g = sems[:NBUF]
        sem_s = sems[NBUF:]
        wid = lax.axis_index("s") * NUM_CORES + lax.axis_index("c")
        base = wid * rows_w
        pltpu.sync_copy(ids_hbm.at[pl.ds(base * C, rows_w * C)], idx_v)

        def gather_pair(j, b):
            idx = idx_v.at[pl.ds(j * G, G)]
            return w_hbm.at[idx], rows_v.at[b]

        def start_gather(j, b):
            src, dst = gather_pair(j, b)
            pltpu.async_copy(src, dst, sem_g[b])

        def wait_gather(j, b):
            src, dst = gather_pair(j, b)
            pltpu.make_async_copy(src, dst, sem_g[b]).wait()

        def store_pairs(j, b):
            for r in range(TOK_R):
                yield (rows_v.at[b].at[pl.ds(r * C, C)],
                       out_hbm.at[base + j * TOK_R + r,
                                  pl.ds(0, C), pl.ds(0, D)])

        def start_store(j, b):
            for src, dst in store_pairs(j, b):
                pltpu.async_copy(src, dst, sem_s[b])

        def wait_store(j, b):
            for src, dst in store_pairs(j, b):
                pltpu.make_async_copy(src, dst, sem_s[b]).wait()

        def body(j, b, bf, first, last):
            if not first:
                wait_store(j - S, bf)
            if not last:
                start_gather(j + K, bf)
            wait_gather(j, b)
            start_store(j, b)

        for j in range(K):
            start_gather(j, j % NBUF)

        for j in range(S):
            body(j, j % NBUF, (j + K) % NBUF, first=True, last=False)

        n_steady = n - K - S
        n_groups, leftover = divmod(n_steady, NBUF)

        @pl.loop(0, n_groups)
        def _group(g):
            for i in range(NBUF):
                b = (S + i) % NBUF
                j = S + g * NBUF + i
                body(j, b, (S + i + K) % NBUF, first=False, last=False)

        for i in range(leftover):
            j = S + n_groups * NBUF + i
            body(j, j % NBUF, (j + K) % NBUF, first=False, last=False)

        for i in range(K):
            j = n - K + i
            body(j, j % NBUF, (j + K) % NBUF, first=False, last=True)

        for i in range(S):
            j = n - S + i
            wait_store(j, j % NBUF)

    return lookup


def kernel(token_ids, W):
    R, C = token_ids.shape
    D = W.shape[1]
    ids_flat = token_ids.reshape(-1).astype(jnp.int32)
    out_padded = _make_lookup(R, C, D)(ids_flat, W)
    return out_padded[:, :C, :D]


# speedup vs baseline: 1.4254x; 1.0005x over previous
import functools

import jax
import jax.numpy as jnp
from jax import lax
from jax.experimental import pallas as pl
from jax.experimental.pallas import tpu as pltpu
from jax.experimental.pallas import tpu_sc as plsc

NUM_CORES = 2
NUM_SUBCORES = 16
NW = NUM_CORES * NUM_SUBCORES

TOK_R = 8
K = 3
S = 1
NBUF = K + S

PAD_C = 56
PAD_D = 128


def _make_lookup(R, C, D):
    assert R % (NW * TOK_R) == 0
    rows_w = R // NW
    n = rows_w // TOK_R
    G = TOK_R * C
    mesh = plsc.VectorSubcoreMesh(core_axis_name="c", subcore_axis_name="s")

    @functools.partial(
        pl.kernel,
        out_type=jax.ShapeDtypeStruct((R, PAD_C, PAD_D), jnp.float32),
        mesh=mesh,
        scratch_types=(
            [pltpu.VMEM((rows_w * C,), jnp.int32),
             pltpu.VMEM((NBUF, G, D), jnp.float32)]
            + [pltpu.SemaphoreType.DMA] * (2 * NBUF)
        ),
        compiler_params=pltpu.CompilerParams(
            use_tc_tiling_on_sc=False, needs_layout_passes=False),
    )
    def lookup(ids_hbm, w_hbm, out_hbm, idx_v, rows_v, *sems):
        sem_g = sems[:NBUF]
        sem_s = sems[NBUF:]
        wid = lax.axis_index("s") * NUM_CORES + lax.axis_index("c")
        base = wid * rows_w
        pltpu.sync_copy(ids_hbm.at[pl.ds(base * C, rows_w * C)], idx_v)

        def gather_pair(j, b):
            idx = idx_v.at[pl.ds(j * G, G)]
            return w_hbm.at[idx], rows_v.at[b]

        def start_gather(j, b):
            src, dst = gather_pair(j, b)
            pltpu.async_copy(src, dst, sem_g[b])

        def wait_gather(j, b):
            src, dst = gather_pair(j, b)
            pltpu.make_async_copy(src, dst, sem_g[b]).wait()

        def store_pairs(j, b):
            for r in range(TOK_R):
                yield (rows_v.at[b].at[pl.ds(r * C, C)],
                       out_hbm.at[base + j * TOK_R + r,
                                  pl.ds(0, C), pl.ds(0, D)])

        def start_store(j, b):
            for src, dst in store_pairs(j, b):
                pltpu.async_copy(src, dst, sem_s[b])

        def wait_store(j, b):
            for src, dst in store_pairs(j, b):
                pltpu.make_async_copy(src, dst, sem_s[b]).wait()

        def body(j, b, bf, first, last):
            if not first:
                wait_store(j - S, bf)
            if not last:
                start_gather(j + K, bf)
            wait_gather(j, b)
            start_store(j, b)

        for j in range(K):
            start_gather(j, j % NBUF)

        for j in range(S):
            body(j, j % NBUF, (j + K) % NBUF, first=True, last=False)

        n_steady = n - K - S
        n_groups, leftover = divmod(n_steady, NBUF)

        @pl.loop(0, n_groups)
        def _group(g):
            for i in range(NBUF):
                b = (S + i) % NBUF
                j = S + g * NBUF + i
                body(j, b, (S + i + K) % NBUF, first=False, last=False)

        for i in range(leftover):
            j = S + n_groups * NBUF + i
            body(j, j % NBUF, (j + K) % NBUF, first=False, last=False)

        for i in range(K):
            j = n - K + i
            body(j, j % NBUF, (j + K) % NBUF, first=False, last=True)

        for i in range(S):
            j = n - S + i
            wait_store(j, j % NBUF)

    return lookup


def kernel(token_ids, W):
    R, C = token_ids.shape
    D = W.shape[1]
    ids_flat = token_ids.reshape(-1).astype(jnp.int32)
    out_padded = _make_lookup(R, C, D)(ids_flat, W)
    return out_padded[:, :C, :D]


# speedup vs baseline: 1.4343x; 1.0063x over previous
import functools

import jax
import jax.numpy as jnp
from jax import lax
from jax.experimental import pallas as pl
from jax.experimental.pallas import tpu as pltpu
from jax.experimental.pallas import tpu_sc as plsc

NUM_CORES = 2
NUM_SUBCORES = 16
NW = NUM_CORES * NUM_SUBCORES

TOK_R = 4
K = 4
S = 2
NBUF = K + S

PAD_C = 56
PAD_D = 128


def _make_lookup(R, C, D):
    assert R % (NW * TOK_R) == 0
    rows_w = R // NW
    n = rows_w // TOK_R
    G = TOK_R * C
    mesh = plsc.VectorSubcoreMesh(core_axis_name="c", subcore_axis_name="s")

    @functools.partial(
        pl.kernel,
        out_type=jax.ShapeDtypeStruct((R, PAD_C, PAD_D), jnp.float32),
        mesh=mesh,
        scratch_types=(
            [pltpu.VMEM((rows_w * C,), jnp.int32),
             pltpu.VMEM((NBUF, G, D), jnp.float32)]
            + [pltpu.SemaphoreType.DMA] * (2 * NBUF)
        ),
        compiler_params=pltpu.CompilerParams(
            use_tc_tiling_on_sc=False, needs_layout_passes=False),
    )
    def lookup(ids_hbm, w_hbm, out_hbm, idx_v, rows_v, *sems):
        sem_g = sems[:NBUF]
        sem_s = sems[NBUF:]
        wid = lax.axis_index("s") * NUM_CORES + lax.axis_index("c")
        base = wid * rows_w
        pltpu.sync_copy(ids_hbm.at[pl.ds(base * C, rows_w * C)], idx_v)

        def gather_pair(j, b):
            idx = idx_v.at[pl.ds(j * G, G)]
            return w_hbm.at[idx], rows_v.at[b]

        def start_gather(j, b):
            src, dst = gather_pair(j, b)
            pltpu.async_copy(src, dst, sem_g[b])

        def wait_gather(j, b):
            src, dst = gather_pair(j, b)
            pltpu.make_async_copy(src, dst, sem_g[b]).wait()

        def store_pairs(j, b):
            for r in range(TOK_R):
                yield (rows_v.at[b].at[pl.ds(r * C, C)],
                       out_hbm.at[base + j * TOK_R + r,
                                  pl.ds(0, C), pl.ds(0, D)])

        def start_store(j, b):
            for src, dst in store_pairs(j, b):
                pltpu.async_copy(src, dst, sem_s[b])

        def wait_store(j, b):
            for src, dst in store_pairs(j, b):
                pltpu.make_async_copy(src, dst, sem_s[b]).wait()

        def body(j, b, bf, first, last):
            if not first:
                wait_store(j - S, bf)
            if not last:
                start_gather(j + K, bf)
            wait_gather(j, b)
            start_store(j, b)

        for j in range(K):
            start_gather(j, j % NBUF)

        for j in range(S):
            body(j, j % NBUF, (j + K) % NBUF, first=True, last=False)

        n_steady = n - K - S
        n_groups, leftover = divmod(n_steady, NBUF)

        @pl.loop(0, n_groups)
        def _group(g):
            for i in range(NBUF):
                b = (S + i) % NBUF
                j = S + g * NBUF + i
                body(j, b, (S + i + K) % NBUF, first=False, last=False)

        for i in range(leftover):
            j = S + n_groups * NBUF + i
            body(j, j % NBUF, (j + K) % NBUF, first=False, last=False)

        for i in range(K):
            j = n - K + i
            body(j, j % NBUF, (j + K) % NBUF, first=False, last=True)

        for i in range(S):
            j = n - S + i
            wait_store(j, j % NBUF)

    return lookup


def kernel(token_ids, W):
    R, C = token_ids.shape
    D = W.shape[1]
    ids_flat = token_ids.reshape(-1).astype(jnp.int32)
    out_padded = _make_lookup(R, C, D)(ids_flat, W)
    return out_padded[:, :C, :D]
